# baseline (speedup 1.0000x reference)
import jax
import jax.numpy as jnp
from jax.experimental import pallas as pl
from jax.experimental.pallas import tpu as pltpu

_SEARCH = 5
_PRE = 7
_KNN = 9
_CIN = 5
_STEM = 32
_BH = 8
_BW = 128
_HALO_H = 2 * _BH
_HALO_W = _BW + 128
_CENTER = (_SEARCH * _SEARCH - 1) // 2
_NROWS = 48
_GROUPS = ((0, 1, 2), (3, 4))


def _tree_argmin(slots, d):
    nodes = [(d[s], None, s) for s in slots]
    while len(nodes) > 1:
        nxt = []
        for a in range(0, len(nodes) - 1, 2):
            vl, il, cl = nodes[a]
            vr, ir, cr = nodes[a + 1]
            lt = vr < vl
            v = jnp.where(lt, vr, vl)
            ilv = jnp.full_like(vl, jnp.float32(cl)) if il is None else il
            irv = jnp.full_like(vr, jnp.float32(cr)) if ir is None else ir
            nxt.append((v, jnp.where(lt, irv, ilv), None))
        if len(nodes) % 2:
            nxt.append(nodes[-1])
        nodes = nxt
    v, i, c = nodes[0]
    return jnp.full_like(v, jnp.float32(c)) if i is None else i


_S_SLOTS = [s for s in range(_SEARCH * _SEARCH) if s != _CENTER]
_P_SLOTS = list(range(_PRE * _PRE))


class _Tile:
    def __init__(self, xq, pq, gi, gp, d1, d2, a1, a2):
        self.xq = xq
        self.pq = pq
        self.gi, self.gp, self.d1, self.d2, self.a1, self.a2 = gi, gp, d1, d2, a1, a2

    def halo(self, frame, c):
        q = self.xq if frame == 0 else self.pq
        top = jnp.concatenate([q[0, 0][0, c], q[0, 1][0, c]], axis=1)
        bot = jnp.concatenate([q[1, 0][0, c], q[1, 1][0, c]], axis=1)
        return jnp.concatenate([top, bot], axis=0)


def _col(hal, dj):
    return jax.lax.slice(hal, (0, dj), (_HALO_H, dj + _BW))


def _rows(shifted, di):
    return jax.lax.slice(shifted, (di, 0), (di + _BH, _BW))


def _zeros():
    return jnp.zeros((_BH, _BW), jnp.float32)


def _diffs(tile, frame, k, roff, center, skip, d_scr):
    hal0 = tile.halo(frame, 0)
    for dj in range(k):
        shifted = _col(hal0, dj + roff)
        for di in range(k):
            s = di * k + dj
            if s == skip:
                continue
            d_scr[s] = jnp.abs(_rows(shifted, di + roff) - center)


def _topk_pass(slots, j, npass, d_scr, a_scr):
    d = {s: d_scr[s] for s in slots}
    ai = _tree_argmin(slots, d)
    a_scr[j] = ai
    if j + 1 < npass:
        for s in slots:
            d_scr[s] = jnp.where(ai == jnp.float32(s), jnp.float32(jnp.inf), d[s])


def _gather(tile, frame, k, roff, npass, jbase, scr, skip, a_scr):
    for group in _GROUPS:
        hals = {c: tile.halo(frame, c) for c in group}
        if skip is not None:
            for c in group:
                scr[c * _KNN] = _rows(_col(hals[c], 2 + roff), 2 + roff)
        ais = [a_scr[j] for j in range(npass)]
        acc = {}
        for dj in range(k):
            shifted = {c: _col(hals[c], dj + roff) for c in group}
            for di in range(k):
                s = di * k + dj
                if s == skip:
                    continue
                vals = {c: _rows(shifted[c], di + roff) for c in group}
                for j in range(npass):
                    m = ais[j] == jnp.float32(s)
                    for c in group:
                        prev = acc.get((c, j))
                        acc[c, j] = jnp.where(m, vals[c], _zeros() if prev is None else prev)
        for (c, j), v in acc.items():
            scr[c * _KNN + jbase + j] = v


def _matmul_out(scr, wref, oref, t):
    z = _zeros()
    scr[45] = z
    scr[46] = z
    scr[47] = z
    g = scr[...].reshape(_NROWS, _BH * _BW)
    o = jnp.maximum(jnp.dot(wref[...], g, preferred_element_type=jnp.float32), 0.0)
    oref[0, :, :, t * _BW:(t + 1) * _BW] = o.reshape(_STEM, _BH, _BW)


def _geometry_j(tile, j, ax, ay, az):
    gp = tile.gp
    x0 = gp[1 * _KNN + j] - ax
    y0 = gp[2 * _KNN + j] - ay
    z0 = gp[3 * _KNN + j] - az
    xy = x0 * x0 + y0 * y0
    z2 = z0 * z0
    r = jnp.sqrt(xy + z2)
    t = jnp.arctan2(jnp.sqrt(xy), z2)
    gp[1 * _KNN + j] = r
    gp[2 * _KNN + j] = t
    gp[3 * _KNN + j] = jnp.arctan2(t * t, r * r)


def _body(*refs):
    xqs = refs[0:6]
    pqs = refs[6:12]
    w_ref, pw_ref, out_ref, pout_ref = refs[12:16]
    scrs = refs[16:]

    def quad(base, t):
        return {(dh, dw): base[dh * 3 + (t + dw)] for dh in range(2) for dw in range(2)}

    tiles = [
        _Tile(quad(xqs, t), quad(pqs, t), *scrs[t * 6:(t + 1) * 6])
        for t in range(2)
    ]

    centers = []
    for t, tile in enumerate(tiles):
        center = _rows(_col(tile.halo(0, 0), 3), 3)
        centers.append(center)
        _diffs(tile, 0, _SEARCH, 1, center, _CENTER, tile.d1)
        _diffs(tile, 1, _PRE, 0, center, None, tile.d2)

    for j in range(_KNN):
        for t, tile in enumerate(tiles):
            if j < _KNN - 1:
                _topk_pass(_S_SLOTS, j, _KNN - 1, tile.d1, tile.a1)
            _topk_pass(_P_SLOTS, j, _KNN, tile.d2, tile.a2)

    for t, tile in enumerate(tiles):
        _gather(tile, 0, _SEARCH, 1, _KNN - 1, 1, tile.gi, _CENTER, tile.a1)
        _matmul_out(tile.gi, w_ref, out_ref, t)

    for t, tile in enumerate(tiles):
        _gather(tile, 1, _PRE, 0, _KNN, 0, tile.gp, None, tile.a2)

    anchors = [(tile.gi[1 * _KNN], tile.gi[2 * _KNN], tile.gi[3 * _KNN])
               for tile in tiles]
    for j in range(_KNN):
        for t, tile in enumerate(tiles):
            _geometry_j(tile, j, *anchors[t])
    for t, tile in enumerate(tiles):
        _matmul_out(tile.gp, pw_ref, pout_ref, t)


def kernel(x, pre_x, range_weight, pre_range_weight):
    B, C, H, W = x.shape
    pad = (_PRE - 1) // 2
    hpad2 = _HALO_H - pad
    wpad2 = _HALO_W + 128 - _BW - pad
    xp = jnp.pad(x, ((0, 0), (0, 0), (pad, hpad2), (pad, wpad2)))
    pxp = jnp.pad(pre_x, ((0, 0), (0, 0), (pad, hpad2), (pad, wpad2)))
    w1 = jnp.pad(range_weight.reshape(_STEM, _CIN * _KNN), ((0, 0), (0, _NROWS - _CIN * _KNN)))
    w2 = jnp.pad(pre_range_weight.reshape(_STEM, _CIN * _KNN), ((0, 0), (0, _NROWS - _CIN * _KNN)))

    grid = (B, H // _BH, W // (2 * _BW))
    out_sds = jax.ShapeDtypeStruct((B, _STEM, H, W), jnp.float32)
    qblock = (1, _CIN, _BH, _BW)

    def qmap(dh, dw):
        return lambda b, h, w: (b, 0, h + dh, 2 * w + dw)

    quad_maps = [qmap(dh, dw) for dh in range(2) for dw in range(3)]
    in_specs = (
        [pl.BlockSpec(qblock, m) for m in quad_maps]
        + [pl.BlockSpec(qblock, m) for m in quad_maps]
        + [
            pl.BlockSpec(w1.shape, lambda b, h, w: (0, 0)),
            pl.BlockSpec(w2.shape, lambda b, h, w: (0, 0)),
        ]
    )
    out_specs = [
        pl.BlockSpec((1, _STEM, _BH, 2 * _BW), lambda b, h, w: (b, 0, h, w)),
        pl.BlockSpec((1, _STEM, _BH, 2 * _BW), lambda b, h, w: (b, 0, h, w)),
    ]
    tile_scratch = [
        pltpu.VMEM((_NROWS, _BH, _BW), jnp.float32),
        pltpu.VMEM((_NROWS, _BH, _BW), jnp.float32),
        pltpu.VMEM((_SEARCH * _SEARCH, _BH, _BW), jnp.float32),
        pltpu.VMEM((_PRE * _PRE, _BH, _BW), jnp.float32),
        pltpu.VMEM((_KNN, _BH, _BW), jnp.float32),
        pltpu.VMEM((_KNN, _BH, _BW), jnp.float32),
    ]
    out, pre_out = pl.pallas_call(
        _body,
        grid=grid,
        in_specs=in_specs,
        out_specs=out_specs,
        out_shape=[out_sds, out_sds],
        scratch_shapes=tile_scratch + tile_scratch,
    )(xp, xp, xp, xp, xp, xp, pxp, pxp, pxp, pxp, pxp, pxp, w1, w2)
    return (out, pre_out)


# speedup vs baseline: 2.0145x; 2.0145x over previous
import jax
import jax.numpy as jnp
from jax.experimental import pallas as pl
from jax.experimental.pallas import tpu as pltpu

_SEARCH = 5
_PRE = 7
_KNN = 9
_CIN = 5
_STEM = 32
_BH = 8
_BW = 128
_CENTER = (_SEARCH * _SEARCH - 1) // 2
_NROWS = 48
_GROUPS = ((0, 1, 2), (3, 4))


def _tree_argmin(slots, d):
    nodes = [(d[s], None, s) for s in slots]
    while len(nodes) > 1:
        nxt = []
        for a in range(0, len(nodes) - 1, 2):
            vl, il, cl = nodes[a]
            vr, ir, cr = nodes[a + 1]
            lt = vr < vl
            v = jnp.where(lt, vr, vl)
            ilv = jnp.full_like(vl, jnp.float32(cl)) if il is None else il
            irv = jnp.full_like(vr, jnp.float32(cr)) if ir is None else ir
            nxt.append((v, jnp.where(lt, irv, ilv), None))
        if len(nodes) % 2:
            nxt.append(nodes[-1])
        nodes = nxt
    v, i, c = nodes[0]
    return jnp.full_like(v, jnp.float32(c)) if i is None else i


def _body(*refs):
    xb = refs[0:2 * _SEARCH]
    pb = refs[2 * _SEARCH:2 * _SEARCH + 2 * _PRE]
    (w_ref, pw_ref, out_ref, pout_ref,
     gi_scr, gp_scr, d1_scr, d2_scr, a1_scr, a2_scr) = refs[2 * _SEARCH + 2 * _PRE:]

    def xcol(c, dj):
        i = dj * 2
        return jnp.concatenate([xb[i][0, 0, c], xb[i + 1][0, 0, c]], axis=0)

    def pcol(c, dj):
        i = dj * 2
        return jnp.concatenate([pb[i][0, 0, c], pb[i + 1][0, 0, c]], axis=0)

    def rows(shifted, di):
        return jax.lax.slice(shifted, (di, 0), (di + _BH, _BW))

    zeros = jnp.zeros((_BH, _BW), jnp.float32)

    center = rows(xcol(0, 2), 3)
    for dj in range(_SEARCH):
        sc = xcol(0, dj)
        for di in range(_SEARCH):
            s = di * _SEARCH + dj
            if s == _CENTER:
                continue
            d1_scr[s] = jnp.abs(rows(sc, di + 1) - center)
    for dj in range(_PRE):
        pc = pcol(0, dj)
        for di in range(_PRE):
            d2_scr[di * _PRE + dj] = jnp.abs(rows(pc, di) - center)

    s_slots = [s for s in range(_SEARCH * _SEARCH) if s != _CENTER]
    p_slots = list(range(_PRE * _PRE))

    def topk_pass(slots, j, npass, d_scr, a_scr):
        d = {s: d_scr[s] for s in slots}
        ai = _tree_argmin(slots, d)
        a_scr[j] = ai
        if j + 1 < npass:
            for s in slots:
                d_scr[s] = jnp.where(ai == jnp.float32(s), jnp.float32(jnp.inf), d[s])

    for j in range(_KNN):
        if j < _KNN - 1:
            topk_pass(s_slots, j, _KNN - 1, d1_scr, a1_scr)
        topk_pass(p_slots, j, _KNN, d2_scr, a2_scr)

    def gather(colf, k, roff, npass, jbase, scr, skip, a_scr):
        for group in _GROUPS:
            if skip is not None:
                for c in group:
                    scr[c * _KNN] = rows(colf(c, 2), 2 + roff)
            ais = [a_scr[j] for j in range(npass)]
            acc = {}
            for dj in range(k):
                shifted = {c: colf(c, dj) for c in group}
                for di in range(k):
                    s = di * k + dj
                    if s == skip:
                        continue
                    vals = {c: rows(shifted[c], di + roff) for c in group}
                    for j in range(npass):
                        m = ais[j] == jnp.float32(s)
                        for c in group:
                            prev = acc.get((c, j))
                            acc[c, j] = jnp.where(m, vals[c], zeros if prev is None else prev)
            for (c, j), v in acc.items():
                scr[c * _KNN + jbase + j] = v

    def matmul_out(scr, wref, oref):
        scr[45] = zeros
        scr[46] = zeros
        scr[47] = zeros
        g = scr[...].reshape(_NROWS, _BH * _BW)
        o = jnp.maximum(jnp.dot(wref[...], g, preferred_element_type=jnp.float32), 0.0)
        oref[0] = o.reshape(_STEM, _BH, _BW)

    gather(xcol, _SEARCH, 1, _KNN - 1, 1, gi_scr, _CENTER, a1_scr)
    matmul_out(gi_scr, w_ref, out_ref)
    gather(pcol, _PRE, 0, _KNN, 0, gp_scr, None, a2_scr)

    ax = gi_scr[1 * _KNN]
    ay = gi_scr[2 * _KNN]
    az = gi_scr[3 * _KNN]
    for j in range(_KNN):
        x0 = gp_scr[1 * _KNN + j] - ax
        y0 = gp_scr[2 * _KNN + j] - ay
        z0 = gp_scr[3 * _KNN + j] - az
        xy = x0 * x0 + y0 * y0
        z2 = z0 * z0
        r = jnp.sqrt(xy + z2)
        t = jnp.arctan2(jnp.sqrt(xy), z2)
        gp_scr[1 * _KNN + j] = r
        gp_scr[2 * _KNN + j] = t
        gp_scr[3 * _KNN + j] = jnp.arctan2(t * t, r * r)
    matmul_out(gp_scr, pw_ref, pout_ref)


def kernel(x, pre_x, range_weight, pre_range_weight):
    B, C, H, W = x.shape
    pad = (_PRE - 1) // 2
    hp = H + 2 * _BH
    wp = W + _BW
    xq = jnp.pad(x, ((0, 0), (0, 0), (pad, hp - H - pad), (pad, wp + _PRE - 1 - W - pad)))
    pq = jnp.pad(pre_x, ((0, 0), (0, 0), (pad, hp - H - pad), (pad, wp + _PRE - 1 - W - pad)))
    xsh = jnp.stack([xq[:, :, :, dj:dj + wp] for dj in range(1, _SEARCH + 1)])
    psh = jnp.stack([pq[:, :, :, dj:dj + wp] for dj in range(_PRE)])
    w1 = jnp.pad(range_weight.reshape(_STEM, _CIN * _KNN), ((0, 0), (0, _NROWS - _CIN * _KNN)))
    w2 = jnp.pad(pre_range_weight.reshape(_STEM, _CIN * _KNN), ((0, 0), (0, _NROWS - _CIN * _KNN)))

    grid = (B, H // _BH, W // _BW)
    out_sds = jax.ShapeDtypeStruct((B, _STEM, H, W), jnp.float32)
    qblock = (1, 1, _CIN, _BH, _BW)

    def qmap(dji, dh):
        return lambda b, h, w: (dji, b, 0, h + dh, w)

    in_specs = (
        [pl.BlockSpec(qblock, qmap(dji, dh)) for dji in range(_SEARCH) for dh in range(2)]
        + [pl.BlockSpec(qblock, qmap(dji, dh)) for dji in range(_PRE) for dh in range(2)]
        + [
            pl.BlockSpec(w1.shape, lambda b, h, w: (0, 0)),
            pl.BlockSpec(w2.shape, lambda b, h, w: (0, 0)),
        ]
    )
    out_specs = [
        pl.BlockSpec((1, _STEM, _BH, _BW), lambda b, h, w: (b, 0, h, w)),
        pl.BlockSpec((1, _STEM, _BH, _BW), lambda b, h, w: (b, 0, h, w)),
    ]
    out, pre_out = pl.pallas_call(
        _body,
        grid=grid,
        in_specs=in_specs,
        out_specs=out_specs,
        out_shape=[out_sds, out_sds],
        scratch_shapes=[
            pltpu.VMEM((_NROWS, _BH, _BW), jnp.float32),
            pltpu.VMEM((_NROWS, _BH, _BW), jnp.float32),
            pltpu.VMEM((_SEARCH * _SEARCH, _BH, _BW), jnp.float32),
            pltpu.VMEM((_PRE * _PRE, _BH, _BW), jnp.float32),
            pltpu.VMEM((_KNN, _BH, _BW), jnp.float32),
            pltpu.VMEM((_KNN, _BH, _BW), jnp.float32),
        ],
    )(*([xsh] * (2 * _SEARCH)), *([psh] * (2 * _PRE)), w1, w2)
    return (out, pre_out)
